# baseline (speedup 1.0000x reference)
import functools

import jax
import jax.numpy as jnp
from jax import lax
from jax.experimental import pallas as pl
from jax.experimental.pallas import tpu as pltpu
from jax.experimental.pallas import tpu_sc as plsc

_NBUF = 2
_LANES = 16


@functools.lru_cache(maxsize=None)
def _make_gather(batch, hist, d):
    info = plsc.get_sparse_core_info()
    num_cores, num_subcores = info.num_cores, info.num_subcores
    num_workers = num_cores * num_subcores
    ent_w = batch // num_workers
    n_r = d // 8
    n_k = ent_w // _LANES
    assert ent_w * num_workers == batch and ent_w == 128
    assert hist % _NBUF == 0 and d % 8 == 0

    mesh = plsc.VectorSubcoreMesh(core_axis_name="c", subcore_axis_name="s")

    @functools.partial(
        pl.kernel,
        out_type=jax.ShapeDtypeStruct(
            (hist, n_r, num_workers, 8, 128), jnp.float32
        ),
        mesh=mesh,
        scratch_types=[
            pltpu.VMEM((ent_w, hist), jnp.int32),
            pltpu.VMEM((hist, ent_w), jnp.int32),
            pltpu.VMEM((_NBUF, ent_w, d), jnp.float32),
            pltpu.VMEM((_NBUF, ent_w, d + 1), jnp.float32),
            pltpu.VMEM((_NBUF, d, 128), jnp.float32),
        ]
        + [pltpu.SemaphoreType.DMA] * (2 * _NBUF),
        compiler_params=pltpu.CompilerParams(
            use_tc_tiling_on_sc=False, needs_layout_passes=False
        ),
    )
    def gather_kernel(
        table_hbm, idx_hbm, out_hbm, idx_v, idx_t, rows_v, rows_p, tbuf, *sems
    ):
        gsem = sems[:_NBUF]
        ssem = sems[_NBUF:]
        wid = lax.axis_index("s") * num_cores + lax.axis_index("c")
        e0 = wid * ent_w
        iota = lax.iota(jnp.int32, _LANES)
        pltpu.sync_copy(idx_hbm.at[pl.ds(e0, ent_w)], idx_v)

        def idx_transpose(h, carry):
            for k in range(n_k):
                vals = plsc.load_gather(
                    idx_v, [iota + _LANES * k, jnp.full((_LANES,), 0, jnp.int32) + h]
                )
                idx_t[h, pl.ds(_LANES * k, _LANES)] = vals
            return carry

        lax.fori_loop(0, hist, idx_transpose, 0)

        def gath(j, b):
            return pltpu.make_async_copy(
                table_hbm.at[idx_t.at[j]], rows_v.at[b], gsem[b]
            )

        def stores(j, b):
            return [
                pltpu.make_async_copy(
                    tbuf.at[b, pl.ds(8 * r, 8)],
                    out_hbm.at[j, r, wid],
                    ssem[b],
                )
                for r in range(n_r)
            ]

        for b in range(_NBUF):
            gath(b, b).start()

        def outer(g, carry):
            for b in range(_NBUF):
                j = g * _NBUF + b
                gath(j, b).wait()

                @pl.when(j >= _NBUF)
                def _():
                    for c in stores(j - _NBUF, b):
                        c.wait()

                def pass1(i, carry):
                    for dl in range(16):
                        l = i * 16 + dl
                        for g in range(d // _LANES):
                            rows_p[b, l, pl.ds(_LANES * g, _LANES)] = rows_v[
                                b, l, pl.ds(_LANES * g, _LANES)
                            ]
                    return carry

                lax.fori_loop(0, ent_w // 16, pass1, 0)

                def pass2(i, carry):
                    for de in range(8):
                        e = i * 8 + de
                        for k in range(n_k):
                            vals = plsc.load_gather(
                                rows_p.at[b],
                                [
                                    iota + _LANES * k,
                                    jnp.full((_LANES,), 0, jnp.int32) + e,
                                ],
                            )
                            tbuf[b, e, pl.ds(_LANES * k, _LANES)] = vals
                    return carry

                lax.fori_loop(0, d // 8, pass2, 0)
                for c in stores(j, b):
                    c.start()
                nxt = j + _NBUF

                @pl.when(nxt < hist)
                def _():
                    gath(nxt, b).start()

            return carry

        lax.fori_loop(0, hist // _NBUF, outer, 0)
        for b in range(_NBUF):
            for c in stores(hist - _NBUF + b, b):
                c.wait()

    return gather_kernel


def kernel(matrix, node_seq_id, G=0):
    batch, hist = node_seq_id.shape
    d = matrix.shape[1]
    out5 = _make_gather(batch, hist, d)(matrix, node_seq_id)
    return out5.transpose(2, 4, 0, 1, 3).reshape(batch, hist, d)


# speedup vs baseline: 1.2759x; 1.2759x over previous
import functools

import jax
import jax.numpy as jnp
from jax import lax
from jax.experimental import pallas as pl
from jax.experimental.pallas import tpu as pltpu
from jax.experimental.pallas import tpu_sc as plsc

_NBUF = 2
_LANES = 16


@functools.lru_cache(maxsize=None)
def _make_gather(batch, hist, d):
    info = plsc.get_sparse_core_info()
    num_cores, num_subcores = info.num_cores, info.num_subcores
    num_workers = num_cores * num_subcores
    ent_w = batch // num_workers
    n_r = d // 8
    n_k = ent_w // _LANES
    assert ent_w * num_workers == batch and ent_w == 128
    assert hist % _NBUF == 0 and d % 8 == 0

    mesh = plsc.VectorSubcoreMesh(core_axis_name="c", subcore_axis_name="s")

    @functools.partial(
        pl.kernel,
        out_type=jax.ShapeDtypeStruct(
            (hist, n_r, num_workers, 8, 128), jnp.float32
        ),
        mesh=mesh,
        scratch_types=[
            pltpu.VMEM((ent_w, hist), jnp.int32),
            pltpu.VMEM((hist, ent_w), jnp.int32),
            pltpu.VMEM((_NBUF, ent_w, d), jnp.float32),
            pltpu.VMEM((_NBUF, ent_w, d + 1), jnp.float32),
            pltpu.VMEM((_NBUF, d, 128), jnp.float32),
        ]
        + [pltpu.SemaphoreType.DMA] * (2 * _NBUF),
        compiler_params=pltpu.CompilerParams(
            use_tc_tiling_on_sc=False, needs_layout_passes=False
        ),
    )
    def gather_kernel(
        table_hbm, idx_hbm, out_hbm, idx_v, idx_t, rows_v, rows_p, tbuf, *sems
    ):
        gsem = sems[:_NBUF]
        ssem = sems[_NBUF:]
        wid = lax.axis_index("s") * num_cores + lax.axis_index("c")
        e0 = wid * ent_w
        iota = lax.iota(jnp.int32, _LANES)
        pltpu.sync_copy(idx_hbm.at[pl.ds(e0, ent_w)], idx_v)

        def idx_transpose(h, carry):
            for k in range(n_k):
                vals = plsc.load_gather(
                    idx_v, [iota + _LANES * k, jnp.full((_LANES,), 0, jnp.int32) + h]
                )
                idx_t[h, pl.ds(_LANES * k, _LANES)] = vals
            return carry

        lax.fori_loop(0, hist, idx_transpose, 0)

        def gath(j, b):
            return pltpu.make_async_copy(
                table_hbm.at[idx_t.at[j]], rows_v.at[b], gsem[b]
            )

        def stores(j, b):
            return [
                pltpu.make_async_copy(
                    tbuf.at[b, pl.ds(8 * r, 8)],
                    out_hbm.at[j, r, wid],
                    ssem[b],
                )
                for r in range(n_r)
            ]

        for b in range(_NBUF):
            gath(b, b).start()

        def outer(g, carry):
            for b in range(_NBUF):
                j = g * _NBUF + b
                gath(j, b).wait()

                @pl.when(j >= _NBUF)
                def _():
                    for c in stores(j - _NBUF, b):
                        c.wait()

                @plsc.parallel_loop(0, ent_w, unroll=8)
                def _(l):
                    for g in range(d // _LANES):
                        rows_p[b, l, pl.ds(_LANES * g, _LANES)] = rows_v[
                            b, l, pl.ds(_LANES * g, _LANES)
                        ]

                @plsc.parallel_loop(0, d, unroll=4)
                def _(e):
                    for k in range(n_k):
                        vals = plsc.load_gather(
                            rows_p.at[b],
                            [
                                iota + _LANES * k,
                                jnp.full((_LANES,), 0, jnp.int32) + e,
                            ],
                        )
                        tbuf[b, e, pl.ds(_LANES * k, _LANES)] = vals
                for c in stores(j, b):
                    c.start()
                nxt = j + _NBUF

                @pl.when(nxt < hist)
                def _():
                    gath(nxt, b).start()

            return carry

        lax.fori_loop(0, hist // _NBUF, outer, 0)
        for b in range(_NBUF):
            for c in stores(hist - _NBUF + b, b):
                c.wait()

    return gather_kernel


def kernel(matrix, node_seq_id, G=0):
    batch, hist = node_seq_id.shape
    d = matrix.shape[1]
    out5 = _make_gather(batch, hist, d)(matrix, node_seq_id)
    return out5.transpose(2, 4, 0, 1, 3).reshape(batch, hist, d)
